# Initial kernel scaffold; baseline (speedup 1.0000x reference)
import jax
import jax.numpy as jnp
from jax.experimental import pallas as pl


def kernel(hidden_states, gate_w, w13, w2):
    raise NotImplementedError("write your pallas kernel here")


# speedup vs baseline: 1.0282x; 1.0282x over previous
import functools

import jax
import jax.numpy as jnp
from jax.experimental import pallas as pl
from jax.experimental.pallas import tpu as pltpu


def _router_body(hid_ref, gw_ref, comb_ref):
    x = hid_ref[...]
    gw = gw_ref[...]
    logits = jax.lax.dot_general(
        x.astype(jnp.bfloat16), gw.astype(jnp.bfloat16), (((1,), (1,)), ((), ())),
        preferred_element_type=jnp.float32,
    )
    e = logits.shape[1]
    lane = jax.lax.broadcasted_iota(jnp.int32, logits.shape, 1)
    m1 = jnp.max(logits, axis=1, keepdims=True)
    i1 = jnp.argmax(logits, axis=1)[:, None]
    masked = jnp.where(lane == i1, -jnp.inf, logits)
    m2 = jnp.max(masked, axis=1, keepdims=True)
    i2 = jnp.argmax(masked, axis=1)[:, None]
    e2 = jnp.exp(m2 - m1)
    w1 = 1.0 / (1.0 + e2)
    w2 = e2 / (1.0 + e2)
    comb_ref[...] = jnp.where(lane == i1, w1, 0.0) + jnp.where(lane == i2, w2, 0.0)


def _ffn_body(d_ff, bt, hid_ref, comb_ref, w13_ref, w2_ref, out_ref):
    e = pl.program_id(0)
    t = pl.program_id(1)

    @pl.when((e == 0) & (t == 0))
    def _():
        out_ref[...] = jnp.zeros_like(out_ref)

    x = hid_ref[...]
    h = jax.lax.dot_general(x, w13_ref[0], (((1,), (0,)), ((), ())),
                            preferred_element_type=jnp.float32)
    g = h[:, :d_ff]
    u = h[:, d_ff:]
    act = (g * jax.nn.sigmoid(g)) * u
    ecol = jax.lax.broadcasted_iota(jnp.int32, comb_ref.shape, 1)
    c = jnp.sum(jnp.where(ecol == e, comb_ref[...], 0.0), axis=1, keepdims=True)
    act = (act * c).astype(jnp.bfloat16)
    p = jax.lax.dot_general(act, w2_ref[0], (((1,), (0,)), ((), ())),
                            preferred_element_type=jnp.float32)
    out_ref[pl.ds(t * bt, bt), :] += p


def kernel(hidden_states, gate_w, w13, w2):
    t, d_model = hidden_states.shape
    n_exp, _, d_ff2 = w13.shape
    d_ff = d_ff2 // 2

    combine = pl.pallas_call(
        _router_body,
        out_shape=jax.ShapeDtypeStruct((t, n_exp), jnp.float32),
    )(hidden_states, gate_w)

    bt = 128 if t % 128 == 0 else t
    nt = t // bt

    hid_bf = hidden_states.astype(jnp.bfloat16)
    w13_bf = w13.astype(jnp.bfloat16)
    w2_bf = w2.astype(jnp.bfloat16)

    grid = (n_exp, nt)
    out = pl.pallas_call(
        functools.partial(_ffn_body, d_ff, bt),
        grid=grid,
        in_specs=[
            pl.BlockSpec((bt, d_model), lambda e, tb: (tb, 0)),
            pl.BlockSpec((bt, n_exp), lambda e, tb: (tb, 0)),
            pl.BlockSpec((1, d_model, d_ff2), lambda e, tb: (e, 0, 0)),
            pl.BlockSpec((1, d_ff, d_model), lambda e, tb: (e, 0, 0)),
        ],
        out_specs=pl.BlockSpec((t, d_model), lambda e, tb: (0, 0)),
        out_shape=jax.ShapeDtypeStruct((t, d_model), jnp.float32),
    )(hid_bf, combine, w13_bf, w2_bf)
    return out
